# Initial kernel scaffold; baseline (speedup 1.0000x reference)
import jax
import jax.numpy as jnp
from jax.experimental import pallas as pl


def kernel(x, edge, neighbors, W_global, W_local, W_self, bias):
    raise NotImplementedError("write your pallas kernel here")


# speedup vs baseline: 1.8864x; 1.8864x over previous
import functools

import jax
import jax.numpy as jnp
from jax import lax
from jax.experimental import pallas as pl
from jax.experimental.pallas import tpu as pltpu
from jax.experimental.pallas import tpu_sc as plsc

N = 10000
DEG = 32
D = 128

NC = 2
NS = 16
NW = NC * NS

NPW = 320
N_PAD = NW * NPW
CH = 4
RPC = CH * DEG
NCH = NPW // CH
E_PAD = N_PAD * DEG


def _make_sc_gather_sum():
    mesh = plsc.VectorSubcoreMesh(core_axis_name="c", subcore_axis_name="s")

    @functools.partial(
        pl.kernel,
        mesh=mesh,
        out_type=jax.ShapeDtypeStruct((N_PAD, D), jnp.float32),
        scratch_types=[
            pltpu.VMEM((NCH, RPC), jnp.int32),
            pltpu.VMEM((2, RPC, D), jnp.float32),
            pltpu.VMEM((CH, D), jnp.float32),
            pltpu.SemaphoreType.DMA,
            pltpu.SemaphoreType.DMA,
        ],
    )
    def sc_gather_sum(x_hbm, nbr_hbm, out_hbm, idx_v, buf, outb, sem0, sem1):
        cid = lax.axis_index("c")
        sid = lax.axis_index("s")
        wid = sid * NC + cid

        pltpu.sync_copy(nbr_hbm.at[pl.ds(wid * NCH, NCH)], idx_v)

        def gather(chunk, b, sem):
            return pltpu.make_async_copy(
                x_hbm.at[idx_v.at[chunk]], buf.at[b], sem)

        gather(0, 0, sem0).start()
        gather(1, 1, sem1).start()

        def process(chunk, b, sem):
            gather(chunk, b, sem).wait()
            for nd in range(CH):
                base = nd * DEG

                def row_body(r, accs):
                    return tuple(
                        accs[k] + buf[b, base + r, pl.ds(16 * k, 16)]
                        for k in range(8))

                accs = lax.fori_loop(
                    0, DEG, row_body,
                    tuple(jnp.zeros((16,), jnp.float32) for _ in range(8)))
                for k in range(8):
                    outb[nd, pl.ds(16 * k, 16)] = accs[k]
            pltpu.sync_copy(outb,
                            out_hbm.at[pl.ds(wid * NPW + chunk * CH, CH)])

        def pair(p, carry):
            process(2 * p, 0, sem0)

            @pl.when(p < NCH // 2 - 1)
            def _():
                gather(2 * p + 2, 0, sem0).start()

            process(2 * p + 1, 1, sem1)

            @pl.when(p < NCH // 2 - 1)
            def _():
                gather(2 * p + 3, 1, sem1).start()

            return carry

        lax.fori_loop(0, NCH // 2, pair, 0)

    return sc_gather_sum


@functools.cache
def _sc_gather_sum_cached():
    return _make_sc_gather_sum()

BR = 1000


def _tc_block(x_ref, s_ref, wg_ref, wl_ref, ws_ref, b_ref, o_ref):
    xb = x_ref[...]
    wc = wg_ref[...] + ws_ref[...]
    z = lax.dot_general(xb, wc, (((1,), (1,)), ((), ())),
                        preferred_element_type=jnp.float32)
    sb = s_ref[...] * (1.0 / DEG)
    z = z + lax.dot_general(sb, wl_ref[...], (((1,), (1,)), ((), ())),
                            preferred_element_type=jnp.float32)
    z = z + b_ref[...]
    o_ref[...] = jnp.where(z > 0.0, z, jnp.exp(jnp.minimum(z, 0.0)) - 1.0)


def _tc_fuse(x, s_pad, Wg, Wl, Ws, bias2d):
    return pl.pallas_call(
        _tc_block,
        grid=(N // BR,),
        in_specs=[
            pl.BlockSpec((BR, D), lambda i: (i, 0)),
            pl.BlockSpec((BR, D), lambda i: (i, 0)),
            pl.BlockSpec((D, D), lambda i: (0, 0)),
            pl.BlockSpec((D, D), lambda i: (0, 0)),
            pl.BlockSpec((D, D), lambda i: (0, 0)),
            pl.BlockSpec((1, D), lambda i: (0, 0)),
        ],
        out_specs=pl.BlockSpec((BR, D), lambda i: (i, 0)),
        out_shape=jax.ShapeDtypeStruct((N, D), jnp.float32),
    )(x, s_pad, Wg, Wl, Ws, bias2d)


def kernel(x, edge, neighbors, W_global, W_local, W_self, bias):
    pad = E_PAD - neighbors.shape[0]
    nbr = jnp.concatenate(
        [neighbors, jnp.zeros((pad,), jnp.int32)]).reshape(NW * NCH, RPC)
    s_pad = _sc_gather_sum_cached()(x, nbr)
    return _tc_fuse(x, s_pad, W_global, W_local, W_self,
                    bias.reshape(1, D))


# speedup vs baseline: 1.9362x; 1.0264x over previous
import functools

import jax
import jax.numpy as jnp
from jax import lax
from jax.experimental import pallas as pl
from jax.experimental.pallas import tpu as pltpu
from jax.experimental.pallas import tpu_sc as plsc

N = 10000
DEG = 32
D = 128

NC = 2
NS = 16
NW = NC * NS

NPW = 320
N_PAD = NW * NPW
CH = 4
RPC = CH * DEG
NCH = NPW // CH
E_PAD = N_PAD * DEG


def _make_sc_gather_sum():
    mesh = plsc.VectorSubcoreMesh(core_axis_name="c", subcore_axis_name="s")

    NB = 4

    @functools.partial(
        pl.kernel,
        mesh=mesh,
        out_type=jax.ShapeDtypeStruct((N_PAD, D), jnp.float32),
        scratch_types=[
            pltpu.VMEM((NCH, RPC), jnp.int32),
            pltpu.VMEM((NB, RPC, D), jnp.float32),
            pltpu.VMEM((NB, CH, D), jnp.float32),
            pltpu.SemaphoreType.DMA,
            pltpu.SemaphoreType.DMA,
            pltpu.SemaphoreType.DMA,
            pltpu.SemaphoreType.DMA,
            pltpu.SemaphoreType.DMA,
            pltpu.SemaphoreType.DMA,
            pltpu.SemaphoreType.DMA,
            pltpu.SemaphoreType.DMA,
        ],
    )
    def sc_gather_sum(x_hbm, nbr_hbm, out_hbm, idx_v, buf, outb,
                      g0, g1, g2, g3, o0, o1, o2, o3):
        gsems = (g0, g1, g2, g3)
        osems = (o0, o1, o2, o3)
        cid = lax.axis_index("c")
        sid = lax.axis_index("s")
        wid = sid * NC + cid

        pltpu.sync_copy(nbr_hbm.at[pl.ds(wid * NCH, NCH)], idx_v)

        def gather(chunk, b):
            return pltpu.make_async_copy(
                x_hbm.at[idx_v.at[chunk]], buf.at[b], gsems[b])

        def out_copy(chunk, b):
            return pltpu.make_async_copy(
                outb.at[b], out_hbm.at[pl.ds(wid * NPW + chunk * CH, CH)],
                osems[b])

        for b in range(NB):
            gather(b, b).start()

        def process(chunk, b):
            gather(chunk, b).wait()

            @pl.when(chunk >= NB)
            def _():
                out_copy(chunk - NB, b).wait()

            for nd in range(CH):
                base = nd * DEG

                def row_body(r4, accs):
                    accs = list(accs)
                    for rr in range(4):
                        for k in range(8):
                            accs[k] = accs[k] + buf[
                                b, base + r4 * 4 + rr, pl.ds(16 * k, 16)]
                    return tuple(accs)

                accs = lax.fori_loop(
                    0, DEG // 4, row_body,
                    tuple(jnp.zeros((16,), jnp.float32) for _ in range(8)))
                for k in range(8):
                    outb[b, nd, pl.ds(16 * k, 16)] = accs[k]
            out_copy(chunk, b).start()

            @pl.when(chunk + NB < NCH)
            def _():
                gather(chunk + NB, b).start()

        def group(p, carry):
            for b in range(NB):
                process(p * NB + b, b)
            return carry

        lax.fori_loop(0, NCH // NB, group, 0)

        for b in range(NB):
            out_copy(NCH - NB + b, b).wait()

    return sc_gather_sum


@functools.cache
def _sc_gather_sum_cached():
    return _make_sc_gather_sum()

BR = 1000


def _tc_block(x_ref, s_ref, wg_ref, wl_ref, ws_ref, b_ref, o_ref):
    xb = x_ref[...]
    wc = wg_ref[...] + ws_ref[...]
    z = lax.dot_general(xb, wc, (((1,), (1,)), ((), ())),
                        preferred_element_type=jnp.float32)
    sb = s_ref[...] * (1.0 / DEG)
    z = z + lax.dot_general(sb, wl_ref[...], (((1,), (1,)), ((), ())),
                            preferred_element_type=jnp.float32)
    z = z + b_ref[...]
    o_ref[...] = jnp.where(z > 0.0, z, jnp.exp(jnp.minimum(z, 0.0)) - 1.0)


def _tc_fuse(x, s_pad, Wg, Wl, Ws, bias2d):
    return pl.pallas_call(
        _tc_block,
        grid=(N // BR,),
        in_specs=[
            pl.BlockSpec((BR, D), lambda i: (i, 0)),
            pl.BlockSpec((BR, D), lambda i: (i, 0)),
            pl.BlockSpec((D, D), lambda i: (0, 0)),
            pl.BlockSpec((D, D), lambda i: (0, 0)),
            pl.BlockSpec((D, D), lambda i: (0, 0)),
            pl.BlockSpec((1, D), lambda i: (0, 0)),
        ],
        out_specs=pl.BlockSpec((BR, D), lambda i: (i, 0)),
        out_shape=jax.ShapeDtypeStruct((N, D), jnp.float32),
    )(x, s_pad, Wg, Wl, Ws, bias2d)


def kernel(x, edge, neighbors, W_global, W_local, W_self, bias):
    pad = E_PAD - neighbors.shape[0]
    nbr = jnp.concatenate(
        [neighbors, jnp.zeros((pad,), jnp.int32)]).reshape(NW * NCH, RPC)
    s_pad = _sc_gather_sum_cached()(x, nbr)
    return _tc_fuse(x, s_pad, W_global, W_local, W_self,
                    bias.reshape(1, D))
